# baseline (speedup 1.0000x reference)
import functools

import jax
import jax.numpy as jnp
from jax import lax
from jax.experimental import pallas as pl
from jax.experimental.pallas import tpu as pltpu
from jax.experimental.pallas import tpu_sc as plsc

_B, _L, _V, _D, _H, _O = 16, 512, 1000000, 128, 128, 128

_NW = 32
_CHUNK = 128
_TOK = _B * _L
_PER_W = _TOK // _NW
_NCH = _PER_W // _CHUNK


def _sc_gather_body(table_hbm, mask_hbm, wid_hbm, mid_hbm, xw_hbm, xm_hbm,
                    idxw_v, idxm_v, rows_w, rows_m, sem):
    wid = lax.axis_index("s") * 2 + lax.axis_index("c")
    pltpu.sync_copy(wid_hbm.at[wid], idxw_v)
    pltpu.sync_copy(mid_hbm.at[wid], idxm_v)
    cps = []
    for k in range(_NCH):
        cps.append(pltpu.async_copy(table_hbm.at[idxw_v.at[k]], rows_w.at[k], sem))
        cps.append(pltpu.async_copy(mask_hbm.at[idxm_v.at[k]], rows_m.at[k], sem))
    for c in cps:
        c.wait()
    pltpu.sync_copy(rows_w, xw_hbm.at[wid])
    pltpu.sync_copy(rows_m, xm_hbm.at[wid])


@functools.cache
def _sc_gather():
    return pl.kernel(
        _sc_gather_body,
        out_type=[
            jax.ShapeDtypeStruct((_NW, _NCH, _CHUNK, _D), jnp.float32),
            jax.ShapeDtypeStruct((_NW, _NCH, _CHUNK, _D), jnp.float32),
        ],
        mesh=plsc.VectorSubcoreMesh(core_axis_name="c", subcore_axis_name="s"),
        scratch_types=[
            pltpu.VMEM((_NCH, _CHUNK), jnp.int32),
            pltpu.VMEM((_NCH, _CHUNK), jnp.int32),
            pltpu.VMEM((_NCH, _CHUNK, _D), jnp.float32),
            pltpu.VMEM((_NCH, _CHUNK, _D), jnp.float32),
            pltpu.SemaphoreType.DMA,
        ],
    )


def _tc_gcn_body(a_ref, xw_ref, xm_ref, w1_ref, b1_ref, w2_ref, b2_ref, o_ref):
    a = a_ref[0]
    x = xw_ref[0] + xm_ref[0]
    ax = jnp.dot(a, x, preferred_element_type=jnp.float32)
    h = jnp.maximum(
        jnp.dot(ax, w1_ref[...], preferred_element_type=jnp.float32) + b1_ref[...],
        0.0)
    ah = jnp.dot(a, h, preferred_element_type=jnp.float32)
    o_ref[0] = jnp.dot(ah, w2_ref[...], preferred_element_type=jnp.float32) + b2_ref[...]


def _tc_gcn(paris_mat, xw, xm, W1, b1, W2, b2):
    return pl.pallas_call(
        _tc_gcn_body,
        grid=(_B,),
        in_specs=[
            pl.BlockSpec((1, _L, _L), lambda b: (b, 0, 0)),
            pl.BlockSpec((1, _L, _D), lambda b: (b, 0, 0)),
            pl.BlockSpec((1, _L, _D), lambda b: (b, 0, 0)),
            pl.BlockSpec((_D, _H), lambda b: (0, 0)),
            pl.BlockSpec((1, _H), lambda b: (0, 0)),
            pl.BlockSpec((_H, _O), lambda b: (0, 0)),
            pl.BlockSpec((1, _O), lambda b: (0, 0)),
        ],
        out_specs=pl.BlockSpec((1, _L, _O), lambda b: (b, 0, 0)),
        out_shape=jax.ShapeDtypeStruct((_B, _L, _O), jnp.float32),
    )(paris_mat, xw, xm, W1, b1, W2, b2)


def kernel(words2ids, i_mask, paris_mat, w_embedding, mask_embedding, W1, b1, W2, b2):
    wid3 = words2ids.reshape(_NW, _NCH, _CHUNK)
    mid3 = i_mask.reshape(_NW, _NCH, _CHUNK)
    xw4, xm4 = _sc_gather()(w_embedding, mask_embedding, wid3, mid3)
    xw = xw4.reshape(_B, _L, _D)
    xm = xm4.reshape(_B, _L, _D)
    return _tc_gcn(paris_mat, xw, xm,
                   W1, b1.reshape(1, _H), W2, b2.reshape(1, _O))


# speedup vs baseline: 4.5377x; 4.5377x over previous
import functools

import jax
import jax.numpy as jnp
from jax import lax
from jax.experimental import pallas as pl
from jax.experimental.pallas import tpu as pltpu
from jax.experimental.pallas import tpu_sc as plsc

_B, _L, _V, _D, _H, _O = 16, 512, 1000000, 128, 128, 128

_NW = 32
_CHUNK = 128
_TOK = _B * _L
_PER_W = _TOK // _NW
_NCH = _PER_W // _CHUNK


def _sc_gather_body(table_hbm, wid_hbm, xw_hbm, idxw_v, rows_w, sem):
    wid = lax.axis_index("s") * 2 + lax.axis_index("c")
    pltpu.sync_copy(wid_hbm.at[wid], idxw_v)
    cps = [pltpu.async_copy(table_hbm.at[idxw_v.at[k]], rows_w.at[k], sem)
           for k in range(_NCH)]
    for c in cps:
        c.wait()
    pltpu.sync_copy(rows_w, xw_hbm.at[wid])


@functools.cache
def _sc_gather():
    return pl.kernel(
        _sc_gather_body,
        out_type=jax.ShapeDtypeStruct((_NW, _NCH, _CHUNK, _D), jnp.float32),
        mesh=plsc.VectorSubcoreMesh(core_axis_name="c", subcore_axis_name="s"),
        scratch_types=[
            pltpu.VMEM((_NCH, _CHUNK), jnp.int32),
            pltpu.VMEM((_NCH, _CHUNK, _D), jnp.float32),
            pltpu.SemaphoreType.DMA,
        ],
    )


def _tc_gcn_body(mf_ref, a_ref, xw_ref, me0_ref, diff_ref, w1_ref, b1_ref,
                 w2_ref, b2_ref, o_ref):
    a = a_ref[0]
    m = mf_ref[0]
    contrib = lax.dot_general(
        m, diff_ref[...], (((0,), (0,)), ((), ())),
        preferred_element_type=jnp.float32)
    x = xw_ref[0] + me0_ref[...] + contrib
    ax = jnp.dot(a, x, preferred_element_type=jnp.float32)
    h = jnp.maximum(
        jnp.dot(ax, w1_ref[...], preferred_element_type=jnp.float32) + b1_ref[...],
        0.0)
    ah = jnp.dot(a, h, preferred_element_type=jnp.float32)
    o_ref[0] = jnp.dot(ah, w2_ref[...], preferred_element_type=jnp.float32) + b2_ref[...]


def _tc_gcn(mf, paris_mat, xw, me0, diff, W1, b1, W2, b2):
    return pl.pallas_call(
        _tc_gcn_body,
        grid=(_B,),
        in_specs=[
            pl.BlockSpec((1, 1, _L), lambda b: (b, 0, 0)),
            pl.BlockSpec((1, _L, _L), lambda b: (b, 0, 0)),
            pl.BlockSpec((1, _L, _D), lambda b: (b, 0, 0)),
            pl.BlockSpec((1, _D), lambda b: (0, 0)),
            pl.BlockSpec((1, _D), lambda b: (0, 0)),
            pl.BlockSpec((_D, _H), lambda b: (0, 0)),
            pl.BlockSpec((1, _H), lambda b: (0, 0)),
            pl.BlockSpec((_H, _O), lambda b: (0, 0)),
            pl.BlockSpec((1, _O), lambda b: (0, 0)),
        ],
        out_specs=pl.BlockSpec((1, _L, _O), lambda b: (b, 0, 0)),
        out_shape=jax.ShapeDtypeStruct((_B, _L, _O), jnp.float32),
    )(mf, paris_mat, xw, me0, diff, W1, b1, W2, b2)


def kernel(words2ids, i_mask, paris_mat, w_embedding, mask_embedding, W1, b1, W2, b2):
    wid3 = words2ids.reshape(_NW, _NCH, _CHUNK)
    xw = _sc_gather()(w_embedding, wid3).reshape(_B, _L, _D)
    mf = i_mask.astype(jnp.float32).reshape(_B, 1, _L)
    me0 = mask_embedding[0].reshape(1, _D)
    diff = (mask_embedding[1] - mask_embedding[0]).reshape(1, _D)
    return _tc_gcn(mf, paris_mat, xw, me0, diff,
                   W1, b1.reshape(1, _H), W2, b2.reshape(1, _O))
